# Initial kernel scaffold; baseline (speedup 1.0000x reference)
import jax
import jax.numpy as jnp
from jax.experimental import pallas as pl


def kernel(context_c, time_cond, gate_W, gate_b, tg_W, tg_b, W1, b1, W2, b2, Ws1, bs1, Ws2, bs2):
    raise NotImplementedError("write your pallas kernel here")


# speedup vs baseline: 1.8170x; 1.8170x over previous
import functools

import jax
import jax.numpy as jnp
from jax.experimental import pallas as pl
from jax.experimental.pallas import tpu as pltpu

B, L, D, E, TOPK = 2, 2048, 768, 8, 2
H = 4 * D
L3 = L // 3
LP = L - 2 * L3

HH = 8
HC = H // HH


def _gelu_tanh(x):
    return 0.5 * x * (1.0 + jnp.tanh(0.7978845608028654 * (x + 0.044715 * x ** 3)))


def _gate_kernel(x_ref, tc_ref, gw_ref, gb_ref, tgw_ref, tgb_ref,
                 idx_ref, wts_ref):
    x = x_ref[...]
    s_head = jnp.sum(x[:, :L3, :], axis=1)
    s_wrist = jnp.sum(x[:, L3:2 * L3, :], axis=1)
    s_prop = jnp.sum(x[:, 2 * L3:, :], axis=1)
    full_agg = (s_head + s_wrist + s_prop) * (1.0 / L)
    hp_agg = (s_head + s_prop) * (1.0 / (L3 + LP))
    wp_agg = (s_wrist + s_prop) * (1.0 / (L3 + LP))
    gate_in = jnp.concatenate([full_agg, hp_agg, wp_agg], axis=-1)
    logits = jnp.dot(gate_in, gw_ref[...],
                     preferred_element_type=jnp.float32) + gb_ref[...]
    tc = tc_ref[...]
    silu = tc * jax.nn.sigmoid(tc)
    mod = jnp.dot(silu, tgw_ref[...],
                  preferred_element_type=jnp.float32) + tgb_ref[...]
    scale = mod[:, :E]
    shift = mod[:, E:]
    logits = logits * (1.0 + scale) + shift
    m = jnp.max(logits, axis=-1, keepdims=True)
    ex = jnp.exp(logits - m)
    sm = ex / jnp.sum(ex, axis=-1, keepdims=True)
    iota = jax.lax.broadcasted_iota(jnp.int32, (B, E), 1)
    m0 = jnp.max(sm, axis=-1, keepdims=True)
    i0 = jnp.min(jnp.where(sm == m0, iota, E), axis=-1, keepdims=True)
    sm1 = jnp.where(iota == i0, -1.0, sm)
    m1 = jnp.max(sm1, axis=-1, keepdims=True)
    i1 = jnp.min(jnp.where(sm1 == m1, iota, E), axis=-1, keepdims=True)
    denom = m0 + m1 + 1e-8
    idx_ref[...] = jnp.concatenate([i0, i1], axis=-1)
    wts_ref[...] = jnp.concatenate([m0 / denom, m1 / denom], axis=-1)


def _moe_kernel(idx_ref, wts_ref, x_ref,
                w1a_ref, w1b_ref, w1s_ref,
                b1a_ref, b1b_ref, b1s_ref,
                w2a_ref, w2b_ref, w2s_ref,
                b2a_ref, b2b_ref, b2s_ref,
                out_ref):
    b = pl.program_id(0)
    hh = pl.program_id(1)
    x = x_ref[0]
    wa = wts_ref[2 * b]
    wb = wts_ref[2 * b + 1]

    def mlp_chunk(w1_ref, b1_ref, w2_ref):
        w1 = w1_ref[...].reshape(D, HC).astype(jnp.bfloat16)
        w2 = w2_ref[...].reshape(HC, D).astype(jnp.bfloat16)
        h = jnp.dot(x, w1, preferred_element_type=jnp.float32)
        h = h + b1_ref[...].reshape(1, HC)
        g = _gelu_tanh(h).astype(jnp.bfloat16)
        return jnp.dot(g, w2, preferred_element_type=jnp.float32)

    ya = mlp_chunk(w1a_ref, b1a_ref, w2a_ref)
    yb = mlp_chunk(w1b_ref, b1b_ref, w2b_ref)
    ys = mlp_chunk(w1s_ref, b1s_ref, w2s_ref)
    acc = wa * ya + wb * yb + ys

    @pl.when(hh == 0)
    def _init():
        bias = (wa * b2a_ref[...].reshape(1, D)
                + wb * b2b_ref[...].reshape(1, D)
                + b2s_ref[...].reshape(1, D))
        out_ref[0] = acc + bias

    @pl.when(hh != 0)
    def _acc():
        out_ref[0] = out_ref[0] + acc


@functools.partial(jax.jit, static_argnames=())
def kernel(context_c, time_cond, gate_W, gate_b, tg_W, tg_b,
           W1, b1, W2, b2, Ws1, bs1, Ws2, bs2):
    idx, wts = pl.pallas_call(
        _gate_kernel,
        out_shape=(
            jax.ShapeDtypeStruct((B, TOPK), jnp.int32),
            jax.ShapeDtypeStruct((B, TOPK), jnp.float32),
        ),
    )(context_c, time_cond, gate_W, gate_b.reshape(1, E),
      tg_W, tg_b.reshape(1, 2 * E))

    x_bf = context_c.astype(jnp.bfloat16)
    idx_flat = idx.reshape(-1)
    wts_flat = wts.reshape(-1)

    grid = (B, HH)
    out = pl.pallas_call(
        _moe_kernel,
        grid_spec=pltpu.PrefetchScalarGridSpec(
            num_scalar_prefetch=2,
            grid=grid,
            in_specs=[
                pl.BlockSpec((1, L, D), lambda b, hh, idx, wts: (b, 0, 0)),
                pl.BlockSpec((1, D, HC),
                             lambda b, hh, idx, wts: (idx[2 * b], 0, hh)),
                pl.BlockSpec((1, D, HC),
                             lambda b, hh, idx, wts: (idx[2 * b + 1], 0, hh)),
                pl.BlockSpec((D, HC), lambda b, hh, idx, wts: (0, hh)),
                pl.BlockSpec((1, 1, HC),
                             lambda b, hh, idx, wts: (idx[2 * b], 0, hh)),
                pl.BlockSpec((1, 1, HC),
                             lambda b, hh, idx, wts: (idx[2 * b + 1], 0, hh)),
                pl.BlockSpec((1, HC), lambda b, hh, idx, wts: (0, hh)),
                pl.BlockSpec((1, HC, D),
                             lambda b, hh, idx, wts: (idx[2 * b], hh, 0)),
                pl.BlockSpec((1, HC, D),
                             lambda b, hh, idx, wts: (idx[2 * b + 1], hh, 0)),
                pl.BlockSpec((HC, D), lambda b, hh, idx, wts: (hh, 0)),
                pl.BlockSpec((1, 1, D),
                             lambda b, hh, idx, wts: (idx[2 * b], 0, 0)),
                pl.BlockSpec((1, 1, D),
                             lambda b, hh, idx, wts: (idx[2 * b + 1], 0, 0)),
                pl.BlockSpec((1, D), lambda b, hh, idx, wts: (0, 0)),
            ],
            out_specs=pl.BlockSpec((1, L, D), lambda b, hh, idx, wts: (b, 0, 0)),
        ),
        out_shape=jax.ShapeDtypeStruct((B, L, D), jnp.float32),
    )(idx_flat, wts_flat, x_bf,
      W1, W1, Ws1,
      b1.reshape(E, 1, H), b1.reshape(E, 1, H), bs1.reshape(1, H),
      W2, W2, Ws2,
      b2.reshape(E, 1, D), b2.reshape(E, 1, D), bs2.reshape(1, D))
    return out
